# Initial kernel scaffold; baseline (speedup 1.0000x reference)
import jax
import jax.numpy as jnp
from jax.experimental import pallas as pl


def kernel(x, V, W0, b0, rows_w, cols_w, vals_w, rows_b, cols_b, vals_b):
    raise NotImplementedError("write your pallas kernel here")


# speedup vs baseline: 6.8524x; 6.8524x over previous
import functools

import jax
import jax.numpy as jnp
from jax import lax
from jax.experimental import pallas as pl
from jax.experimental.pallas import tpu as pltpu
from jax.experimental.pallas import tpu_sc as plsc

D_MODEL = 2048
ID_DIM = 4096
W_DIM = D_MODEL * D_MODEL

NC = 2
NS = 16
L = 16

NUM_CHUNKS = 4
R_CHUNK = W_DIM // NUM_CHUNKS
BE = 4096


def _sc_scatter_build(nnz_w, eb):
    per_tile_w = nnz_w // NS
    n_blocks = per_tile_w // BE
    stripe = R_CHUNK // NS
    mesh = plsc.VectorSubcoreMesh(core_axis_name="c", subcore_axis_name="s",
                                  num_cores=NC, num_subcores=NS)

    @functools.partial(
        pl.kernel,
        out_type=[
            jax.ShapeDtypeStruct((W_DIM,), jnp.float32),
            jax.ShapeDtypeStruct((NC, D_MODEL), jnp.float32),
        ],
        mesh=mesh,
        compiler_params=pltpu.CompilerParams(needs_layout_passes=False),
        scratch_types=[
            pltpu.VMEM((ID_DIM,), jnp.float32),
            pltpu.VMEM((BE,), jnp.int32),
            pltpu.VMEM((BE,), jnp.int32),
            pltpu.VMEM((BE,), jnp.float32),
            pltpu.VMEM((BE,), jnp.int32),
            pltpu.VMEM((BE,), jnp.float32),
            pltpu.VMEM((eb,), jnp.int32),
            pltpu.VMEM((eb,), jnp.int32),
            pltpu.VMEM((eb,), jnp.float32),
            pltpu.VMEM((eb,), jnp.int32),
            pltpu.VMEM((eb,), jnp.float32),
            pltpu.VMEM_SHARED((R_CHUNK + L,), jnp.float32),
            pltpu.VMEM_SHARED((D_MODEL,), jnp.float32),
        ],
    )
    def sc_kernel(v_hbm, w0_hbm, rows_hbm, cols_hbm, vals_hbm,
                  brows_hbm, bcols_hbm, bvals_hbm, zeros_hbm,
                  w_out, pb_out,
                  v_v, rows_v, cols_v, vals_v, idx_v, val_v,
                  brows_v, bcols_v, bvals_v, bidx_v, bval_v,
                  wacc_s, bacc_s):
        cid = lax.axis_index("c")
        sid = lax.axis_index("s")

        pltpu.sync_copy(v_hbm, v_v)

        @pl.when(sid == 0)
        def _():
            pltpu.sync_copy(zeros_hbm, bacc_s)

        def scan_chunk(base):
            def blk_body(blk, carry):
                estart = sid * per_tile_w + blk * BE
                pltpu.sync_copy(rows_hbm.at[pl.ds(estart, BE)], rows_v)
                pltpu.sync_copy(cols_hbm.at[pl.ds(estart, BE)], cols_v)
                pltpu.sync_copy(vals_hbm.at[pl.ds(estart, BE)], vals_v)

                def grp(g, c):
                    for u in range(4):
                        off = (g * 4 + u) * L
                        r16 = rows_v[pl.ds(off, L)]
                        c16 = cols_v[pl.ds(off, L)]
                        a16 = vals_v[pl.ds(off, L)]
                        vv = plsc.load_gather(v_v, [c16])
                        li = r16 - base
                        inb = (li >= 0) & (li < R_CHUNK)
                        idx_v[pl.ds(off, L)] = jnp.where(inb, li, R_CHUNK)
                        val_v[pl.ds(off, L)] = a16 * vv
                    return c

                lax.fori_loop(0, BE // (4 * L), grp, 0)
                pltpu.sync_copy(val_v, wacc_s.at[idx_v], add=True)
                return carry

            lax.fori_loop(0, n_blocks, blk_body, 0)

        for chunk_i in range(NUM_CHUNKS // NC):
            base = (cid * (NUM_CHUNKS // NC) + chunk_i) * R_CHUNK
            pltpu.sync_copy(w0_hbm.at[pl.ds(base + sid * stripe, stripe)],
                            wacc_s.at[pl.ds(sid * stripe, stripe)])
            plsc.subcore_barrier()
            scan_chunk(base)
            plsc.subcore_barrier()
            pltpu.sync_copy(wacc_s.at[pl.ds(sid * stripe, stripe)],
                            w_out.at[pl.ds(base + sid * stripe, stripe)])
            plsc.subcore_barrier()

        wid = sid * NC + cid
        bstart = wid * eb
        pltpu.sync_copy(brows_hbm.at[pl.ds(bstart, eb)], brows_v)
        pltpu.sync_copy(bcols_hbm.at[pl.ds(bstart, eb)], bcols_v)
        pltpu.sync_copy(bvals_hbm.at[pl.ds(bstart, eb)], bvals_v)

        def bgrp(g, c):
            off = g * L
            r16 = brows_v[pl.ds(off, L)]
            c16 = bcols_v[pl.ds(off, L)]
            a16 = bvals_v[pl.ds(off, L)]
            vv = plsc.load_gather(v_v, [c16])
            bidx_v[pl.ds(off, L)] = r16
            bval_v[pl.ds(off, L)] = a16 * vv
            return c

        lax.fori_loop(0, eb // L, bgrp, 0)
        pltpu.sync_copy(bval_v, bacc_s.at[bidx_v], add=True)
        plsc.subcore_barrier()

        @pl.when(sid == 0)
        def _():
            pltpu.sync_copy(bacc_s, pb_out.at[cid])

    return sc_kernel


def _mm_block(x_ref, w_ref, b0_ref, pb_ref, o_ref):
    b = b0_ref[0] + pb_ref[0] + pb_ref[1]
    o_ref[...] = lax.dot_general(
        x_ref[...], w_ref[...], (((1,), (1,)), ((), ())),
        preferred_element_type=jnp.float32) + b[None, :]


def _matmul(x, w, b0, pb):
    n_tok, d = x.shape
    bm, bn = 1024, 1024
    grid = (d // bn, n_tok // bm)
    return pl.pallas_call(
        _mm_block,
        grid=grid,
        in_specs=[
            pl.BlockSpec((bm, d), lambda j, i: (i, 0)),
            pl.BlockSpec((bn, d), lambda j, i: (j, 0)),
            pl.BlockSpec((1, bn), lambda j, i: (0, j)),
            pl.BlockSpec((NC, bn), lambda j, i: (0, j)),
        ],
        out_specs=pl.BlockSpec((bm, bn), lambda j, i: (i, j)),
        out_shape=jax.ShapeDtypeStruct((n_tok, d), jnp.float32),
    )(x, w, b0, pb)


def kernel(x, V, W0, b0, rows_w, cols_w, vals_w, rows_b, cols_b, vals_b):
    v = V[:, 0]
    nnz_w = rows_w.shape[0]
    nnz_b = rows_b.shape[0]

    nw = NC * NS
    eb = -(-nnz_b // (nw * L)) * L
    pad = nw * eb - nnz_b
    rbp = jnp.concatenate([rows_b, jnp.zeros((pad,), rows_b.dtype)])
    cbp = jnp.concatenate([cols_b, jnp.zeros((pad,), cols_b.dtype)])
    vbp = jnp.concatenate([vals_b, jnp.zeros((pad,), vals_b.dtype)])

    sc = _sc_scatter_build(nnz_w, eb)
    w_full, pb = sc(v, W0.reshape(-1), rows_w, cols_w, vals_w,
                    rbp, cbp, vbp, jnp.zeros((D_MODEL,), jnp.float32))
    return _matmul(x, w_full.reshape(D_MODEL, D_MODEL), b0.reshape(1, -1), pb)


# speedup vs baseline: 68.0677x; 9.9334x over previous
import functools

import jax
import jax.numpy as jnp
from jax import lax
from jax.experimental import pallas as pl
from jax.experimental.pallas import tpu as pltpu
from jax.experimental.pallas import tpu_sc as plsc

D_MODEL = 2048
ID_DIM = 4096
W_DIM = D_MODEL * D_MODEL

NC = 2
NS = 16
L = 16

NUM_CHUNKS = 4
R_CHUNK = W_DIM // NUM_CHUNKS
BE = 4096
TRASH = 4096


def _sc_scatter_build(nnz_w, eb):
    per_tile_w = nnz_w // NS
    n_blocks = per_tile_w // BE
    stripe = R_CHUNK // NS
    mesh = plsc.VectorSubcoreMesh(core_axis_name="c", subcore_axis_name="s",
                                  num_cores=NC, num_subcores=NS)

    @functools.partial(
        pl.kernel,
        out_type=[
            jax.ShapeDtypeStruct((W_DIM,), jnp.float32),
            jax.ShapeDtypeStruct((NC, D_MODEL), jnp.float32),
        ],
        mesh=mesh,
        compiler_params=pltpu.CompilerParams(needs_layout_passes=False),
        scratch_types=[
            pltpu.VMEM((ID_DIM,), jnp.float32),
            pltpu.VMEM((BE,), jnp.int32),
            pltpu.VMEM((BE,), jnp.int32),
            pltpu.VMEM((BE,), jnp.float32),
            pltpu.VMEM((BE,), jnp.int32),
            pltpu.VMEM((BE,), jnp.float32),
            pltpu.VMEM((eb,), jnp.int32),
            pltpu.VMEM((eb,), jnp.int32),
            pltpu.VMEM((eb,), jnp.float32),
            pltpu.VMEM((eb,), jnp.int32),
            pltpu.VMEM((eb,), jnp.float32),
            pltpu.VMEM_SHARED((R_CHUNK + TRASH,), jnp.float32),
            pltpu.VMEM_SHARED((D_MODEL,), jnp.float32),
        ],
    )
    def sc_kernel(v_hbm, w0_hbm, rows_hbm, cols_hbm, vals_hbm,
                  brows_hbm, bcols_hbm, bvals_hbm, zeros_hbm,
                  w_out, pb_out,
                  v_v, rows_v, cols_v, vals_v, idx_v, val_v,
                  brows_v, bcols_v, bvals_v, bidx_v, bval_v,
                  wacc_s, bacc_s):
        cid = lax.axis_index("c")
        sid = lax.axis_index("s")

        pltpu.sync_copy(v_hbm, v_v)

        @pl.when(sid == 0)
        def _():
            pltpu.sync_copy(zeros_hbm, bacc_s)

        def scan_chunk(base):
            def blk_body(blk, carry):
                estart = sid * per_tile_w + blk * BE
                pltpu.sync_copy(rows_hbm.at[pl.ds(estart, BE)], rows_v)
                pltpu.sync_copy(cols_hbm.at[pl.ds(estart, BE)], cols_v)
                pltpu.sync_copy(vals_hbm.at[pl.ds(estart, BE)], vals_v)

                def grp(g, c):
                    for u in range(4):
                        off = (g * 4 + u) * L
                        r16 = rows_v[pl.ds(off, L)]
                        c16 = cols_v[pl.ds(off, L)]
                        a16 = vals_v[pl.ds(off, L)]
                        vv = plsc.load_gather(v_v, [c16])
                        li = r16 - base
                        inb = (li >= 0) & (li < R_CHUNK)
                        trash = R_CHUNK + (r16 & (TRASH - 1))
                        idx_v[pl.ds(off, L)] = jnp.where(inb, li, trash)
                        val_v[pl.ds(off, L)] = a16 * vv
                    return c

                lax.fori_loop(0, BE // (4 * L), grp, 0)
                pltpu.sync_copy(val_v, wacc_s.at[idx_v], add=True)
                return carry

            lax.fori_loop(0, n_blocks, blk_body, 0)

        for chunk_i in range(NUM_CHUNKS // NC):
            base = (cid * (NUM_CHUNKS // NC) + chunk_i) * R_CHUNK
            pltpu.sync_copy(w0_hbm.at[pl.ds(base + sid * stripe, stripe)],
                            wacc_s.at[pl.ds(sid * stripe, stripe)])
            plsc.subcore_barrier()
            scan_chunk(base)
            plsc.subcore_barrier()
            pltpu.sync_copy(wacc_s.at[pl.ds(sid * stripe, stripe)],
                            w_out.at[pl.ds(base + sid * stripe, stripe)])
            plsc.subcore_barrier()

        wid = sid * NC + cid
        bstart = wid * eb
        pltpu.sync_copy(brows_hbm.at[pl.ds(bstart, eb)], brows_v)
        pltpu.sync_copy(bcols_hbm.at[pl.ds(bstart, eb)], bcols_v)
        pltpu.sync_copy(bvals_hbm.at[pl.ds(bstart, eb)], bvals_v)

        def bgrp(g, c):
            off = g * L
            r16 = brows_v[pl.ds(off, L)]
            c16 = bcols_v[pl.ds(off, L)]
            a16 = bvals_v[pl.ds(off, L)]
            vv = plsc.load_gather(v_v, [c16])
            bidx_v[pl.ds(off, L)] = r16
            bval_v[pl.ds(off, L)] = a16 * vv
            return c

        lax.fori_loop(0, eb // L, bgrp, 0)
        pltpu.sync_copy(bval_v, bacc_s.at[bidx_v], add=True)
        plsc.subcore_barrier()

        @pl.when(sid == 0)
        def _():
            pltpu.sync_copy(bacc_s, pb_out.at[cid])

    return sc_kernel


def _mm_block(x_ref, w_ref, b0_ref, pb_ref, o_ref):
    b = b0_ref[0] + pb_ref[0] + pb_ref[1]
    o_ref[...] = lax.dot_general(
        x_ref[...], w_ref[...], (((1,), (1,)), ((), ())),
        preferred_element_type=jnp.float32) + b[None, :]


def _matmul(x, w, b0, pb):
    n_tok, d = x.shape
    bm, bn = 1024, 1024
    grid = (d // bn, n_tok // bm)
    return pl.pallas_call(
        _mm_block,
        grid=grid,
        in_specs=[
            pl.BlockSpec((bm, d), lambda j, i: (i, 0)),
            pl.BlockSpec((bn, d), lambda j, i: (j, 0)),
            pl.BlockSpec((1, bn), lambda j, i: (0, j)),
            pl.BlockSpec((NC, bn), lambda j, i: (0, j)),
        ],
        out_specs=pl.BlockSpec((bm, bn), lambda j, i: (i, j)),
        out_shape=jax.ShapeDtypeStruct((n_tok, d), jnp.float32),
    )(x, w, b0, pb)


def kernel(x, V, W0, b0, rows_w, cols_w, vals_w, rows_b, cols_b, vals_b):
    v = V[:, 0]
    nnz_w = rows_w.shape[0]
    nnz_b = rows_b.shape[0]

    nw = NC * NS
    eb = -(-nnz_b // (nw * L)) * L
    pad = nw * eb - nnz_b
    rbp = jnp.concatenate([rows_b, jnp.zeros((pad,), rows_b.dtype)])
    cbp = jnp.concatenate([cols_b, jnp.zeros((pad,), cols_b.dtype)])
    vbp = jnp.concatenate([vals_b, jnp.zeros((pad,), vals_b.dtype)])

    sc = _sc_scatter_build(nnz_w, eb)
    w_full, pb = sc(v, W0.reshape(-1), rows_w, cols_w, vals_w,
                    rbp, cbp, vbp, jnp.zeros((D_MODEL,), jnp.float32))
    return _matmul(x, w_full.reshape(D_MODEL, D_MODEL), b0.reshape(1, -1), pb)
